# baseline (speedup 1.0000x reference)
import functools
import jax
import jax.numpy as jnp
from jax import lax
from jax.experimental import pallas as pl
from jax.experimental.pallas import tpu as pltpu
from jax.experimental.pallas import tpu_sc as plsc

END_TOK = 49407
B, L, D = 16, 2048, 1024
CH = 32
NCHUNK = 1024 // CH

_mesh = plsc.VectorSubcoreMesh(
    core_axis_name="c", subcore_axis_name="s", num_cores=2, num_subcores=16)


@functools.partial(
    pl.kernel,
    mesh=_mesh,
    out_type=jax.ShapeDtypeStruct((B, L + 1, D), jnp.float32),
    scratch_types=[
        pltpu.VMEM((16,), jnp.int32),
        pltpu.VMEM((CH, D), jnp.float32),
        pltpu.VMEM((CH, D), jnp.float32),
    ],
    compiler_params=pltpu.CompilerParams(needs_layout_passes=False),
)
def _sc_kernel(plen_hbm, vt_hbm, emb_hbm, out_hbm, plen_v, cbuf, fbuf):
    wid = lax.axis_index("s") * 2 + lax.axis_index("c")
    b = wid // 2
    half = wid % 2
    S = half * 1024

    pltpu.sync_copy(plen_hbm, plen_v)
    pltpu.sync_copy(emb_hbm.at[pl.ds(END_TOK, 1)], fbuf.at[pl.ds(0, 1)])

    def _rep(r, carry):
        for c in range(D // 16):
            fbuf[r, pl.ds(c * 16, 16)] = fbuf[0, pl.ds(c * 16, 16)]
        return carry
    lax.fori_loop(1, CH, _rep, 0, unroll=False)

    lanes = lax.iota(jnp.int32, 16)
    p = lax.reduce_max(
        jnp.where(lanes == b, plen_v[...], jnp.int32(0)), axes=(0,))
    n_copy = jnp.clip(p - S, 0, 1024)
    nc = (n_copy + CH - 1) // CH
    rem = n_copy % CH

    def _patch(r, carry):
        for c in range(D // 16):
            cbuf[r, pl.ds(c * 16, 16)] = fbuf[0, pl.ds(c * 16, 16)]
        return carry

    def _chunk(i, carry):
        off = S + i * CH
        @pl.when(i < nc)
        def _():
            pltpu.sync_copy(vt_hbm.at[b, pl.ds(off, CH)], cbuf)
            @pl.when((i == nc - 1) & (rem != 0))
            def _():
                lax.fori_loop(rem, CH, _patch, 0, unroll=False)
            pltpu.sync_copy(cbuf, out_hbm.at[b, pl.ds(off, CH)])
        @pl.when(i >= nc)
        def _():
            pltpu.sync_copy(fbuf, out_hbm.at[b, pl.ds(off, CH)])
        return carry

    lax.fori_loop(0, NCHUNK, _chunk, 0, unroll=False)

    @pl.when(half == 1)
    def _():
        pltpu.sync_copy(fbuf.at[pl.ds(0, 1)], out_hbm.at[b, pl.ds(L, 1)])


def kernel(categories, vt, emb):
    plen = jnp.sum((categories != END_TOK).astype(jnp.int32), axis=1)
    return _sc_kernel(plen, vt, emb)


# speedup vs baseline: 1.0667x; 1.0667x over previous
import functools
import jax
import jax.numpy as jnp
from jax import lax
from jax.experimental import pallas as pl
from jax.experimental.pallas import tpu as pltpu
from jax.experimental.pallas import tpu_sc as plsc

END_TOK = 49407
B, L, D = 16, 2048, 1024
CH = 32
NCHUNK = 1024 // CH
FILL_LAG = 8

_mesh = plsc.VectorSubcoreMesh(
    core_axis_name="c", subcore_axis_name="s", num_cores=2, num_subcores=16)


@functools.partial(
    pl.kernel,
    mesh=_mesh,
    out_type=jax.ShapeDtypeStruct((B, L + 1, D), jnp.float32),
    scratch_types=[
        pltpu.VMEM((16,), jnp.int32),
        pltpu.VMEM((CH, D), jnp.float32),
        pltpu.VMEM((CH, D), jnp.float32),
        pltpu.VMEM((CH, D), jnp.float32),
        pltpu.SemaphoreType.DMA,
        pltpu.SemaphoreType.DMA,
        pltpu.SemaphoreType.DMA,
        pltpu.SemaphoreType.DMA,
        pltpu.SemaphoreType.DMA,
    ],
    compiler_params=pltpu.CompilerParams(needs_layout_passes=False),
)
def _sc_kernel(plen_hbm, vt_hbm, emb_hbm, out_hbm, plen_v, cbuf0, cbuf1,
               fbuf, rsem0, rsem1, wsem0, wsem1, fsem):
    wid = lax.axis_index("s") * 2 + lax.axis_index("c")
    b = wid // 2
    half = wid % 2
    S = half * 1024

    pltpu.sync_copy(plen_hbm, plen_v)
    pltpu.sync_copy(emb_hbm.at[pl.ds(END_TOK, 1)], fbuf.at[pl.ds(0, 1)])

    def _rep(r, carry):
        for c in range(D // 16):
            fbuf[r, pl.ds(c * 16, 16)] = fbuf[0, pl.ds(c * 16, 16)]
        return carry
    lax.fori_loop(1, CH, _rep, 0, unroll=False)

    lanes = lax.iota(jnp.int32, 16)
    p = lax.reduce_max(
        jnp.where(lanes == b, plen_v[...], jnp.int32(0)), axes=(0,))
    n_copy = jnp.clip(p - S, 0, 1024)
    nc = (n_copy + CH - 1) // CH
    rem = n_copy % CH

    def make_patch(buf):
        def _p(r, carry):
            for c in range(D // 16):
                buf[r, pl.ds(c * 16, 16)] = fbuf[0, pl.ds(c * 16, 16)]
            return carry
        return _p

    def read_dma(i, buf, sem):
        return pltpu.make_async_copy(
            vt_hbm.at[b, pl.ds(S + i * CH, CH)], buf, sem)

    def write_dma(i, buf, sem):
        return pltpu.make_async_copy(
            buf, out_hbm.at[b, pl.ds(S + i * CH, CH)], sem)

    def fill_dma(i):
        return pltpu.make_async_copy(
            fbuf, out_hbm.at[b, pl.ds(S + i * CH, CH)], fsem)

    @pl.when(half == 1)
    def _():
        pltpu.async_copy(
            fbuf.at[pl.ds(0, 1)], out_hbm.at[b, pl.ds(L, 1)], fsem)

    def _pair(j, carry):
        c0 = 2 * j
        c1 = 2 * j + 1

        @pl.when((c0 < nc) & (c0 >= 2))
        def _():
            write_dma(c0 - 2, cbuf0, wsem0).wait()
        @pl.when(c0 < nc)
        def _():
            read_dma(c0, cbuf0, rsem0).start()
        @pl.when((c1 < nc) & (c1 >= 2))
        def _():
            write_dma(c1 - 2, cbuf1, wsem1).wait()
        @pl.when(c1 < nc)
        def _():
            read_dma(c1, cbuf1, rsem1).start()

        @pl.when(c0 < nc)
        def _():
            read_dma(c0, cbuf0, rsem0).wait()
            @pl.when((c0 == nc - 1) & (rem != 0))
            def _():
                lax.fori_loop(rem, CH, make_patch(cbuf0), 0, unroll=False)
            write_dma(c0, cbuf0, wsem0).start()
        @pl.when(c0 >= nc)
        def _():
            fill_dma(c0).start()

        @pl.when(c1 < nc)
        def _():
            read_dma(c1, cbuf1, rsem1).wait()
            @pl.when((c1 == nc - 1) & (rem != 0))
            def _():
                lax.fori_loop(rem, CH, make_patch(cbuf1), 0, unroll=False)
            write_dma(c1, cbuf1, wsem1).start()
        @pl.when(c1 >= nc)
        def _():
            fill_dma(c1).start()

        @pl.when(c0 - FILL_LAG >= nc)
        def _():
            fill_dma(c0 - FILL_LAG).wait()
        @pl.when(c1 - FILL_LAG >= nc)
        def _():
            fill_dma(c1 - FILL_LAG).wait()
        return carry

    lax.fori_loop(0, NCHUNK // 2, _pair, 0, unroll=False)

    @pl.when(nc >= 2)
    def _():
        write_dma(0, cbuf0, wsem0).wait()
        write_dma(0, cbuf1, wsem1).wait()
    @pl.when(nc == 1)
    def _():
        write_dma(0, cbuf0, wsem0).wait()

    def _fdrain(c, carry):
        fill_dma(c).wait()
        return carry
    lax.fori_loop(jnp.maximum(nc, NCHUNK - FILL_LAG), NCHUNK, _fdrain, 0,
                  unroll=False)

    @pl.when(half == 1)
    def _():
        pltpu.make_async_copy(
            fbuf.at[pl.ds(0, 1)], out_hbm.at[b, pl.ds(L, 1)], fsem).wait()


def kernel(categories, vt, emb):
    plen = jnp.sum((categories != END_TOK).astype(jnp.int32), axis=1)
    return _sc_kernel(plen, vt, emb)


# speedup vs baseline: 1.3145x; 1.2323x over previous
import jax
import jax.numpy as jnp
from jax.experimental import pallas as pl
from jax.experimental.pallas import tpu as pltpu

END_TOK = 49407


def _prefix_dma(vt_ref, buf_ref, sem, row, c8, do_start):
    for k in range(11, 2, -1):
        size = 1 << k
        off = pl.multiple_of((c8 >> (k + 1)) << (k + 1), size * 2)
        @pl.when((c8 & size) != 0)
        def _():
            dma = pltpu.make_async_copy(
                vt_ref.at[row, pl.ds(off, size)],
                buf_ref.at[pl.ds(off, size)],
                sem,
            )
            dma.start() if do_start else dma.wait()


def _body(plen_ref, perm_ref, vt_ref, end_ref, out_ref, buf0, buf1, sem0,
          sem1):
    B, L, D = vt_ref.shape
    g = pl.program_id(0)

    def row_of(i):
        return perm_ref[i]

    def c8_of(i):
        return pl.multiple_of(
            jnp.minimum((plen_ref[perm_ref[i]] + 7) & ~7, L), 8)

    plen = plen_ref[perm_ref[g]]
    c8 = c8_of(g)
    end_row = end_ref[END_TOK % 8:END_TOK % 8 + 1, :]

    def issue(cur_buf, cur_sem, nxt_buf, nxt_sem):
        @pl.when(g == 0)
        def _():
            _prefix_dma(vt_ref, cur_buf, cur_sem, row_of(0), c8_of(0), True)

        @pl.when(g + 1 < B)
        def _():
            _prefix_dma(vt_ref, nxt_buf, nxt_sem, row_of(g + 1),
                        c8_of(g + 1), True)

    def finish(cur_buf, cur_sem):
        _prefix_dma(vt_ref, cur_buf, cur_sem, row_of(g), c8, False)

        for k in range(11, 2, -1):
            size = 1 << k
            off = pl.multiple_of((c8 >> (k + 1)) << (k + 1), size * 2)
            @pl.when((c8 & size) != 0)
            def _():
                out_ref[0, pl.ds(off, size)] = cur_buf[pl.ds(off, size)]

        @pl.when(c8 > plen)
        def _():
            f8 = pl.multiple_of(c8 - 8, 8)
            rows8 = jax.lax.broadcasted_iota(jnp.int32, (8, 1), 0) + f8
            tile = cur_buf[pl.ds(f8, 8)]
            out_ref[0, pl.ds(f8, 8)] = jnp.where(rows8 < plen, tile,
                                                 end_row)

    @pl.when(g % 2 == 0)
    def _():
        issue(buf0, sem0, buf1, sem1)

    @pl.when(g % 2 == 1)
    def _():
        issue(buf1, sem1, buf0, sem0)

    q = (L - c8) >> 3
    for k in range(8, -1, -1):
        rows = 8 << k
        off = pl.multiple_of(c8 + ((q >> (k + 1)) << (k + 1)) * 8, 8)
        @pl.when((q & (1 << k)) != 0)
        def _():
            out_ref[0, pl.ds(off, rows)] = jnp.broadcast_to(
                end_row, (rows, D))
    out_ref[0, pl.ds(L, 1)] = end_row

    @pl.when(g % 2 == 0)
    def _():
        finish(buf0, sem0)

    @pl.when(g % 2 == 1)
    def _():
        finish(buf1, sem1)


def kernel(categories, vt, emb):
    B, L = categories.shape
    D = vt.shape[-1]
    plen = jnp.sum((categories != END_TOK).astype(jnp.int32), axis=1)
    perm = jnp.argsort(plen).astype(jnp.int32)

    grid_spec = pltpu.PrefetchScalarGridSpec(
        num_scalar_prefetch=2,
        grid=(B,),
        in_specs=[
            pl.BlockSpec(memory_space=pl.ANY),
            pl.BlockSpec((8, D), lambda g, p, pm: (END_TOK // 8, 0)),
        ],
        out_specs=pl.BlockSpec((1, L + 1, D), lambda g, p, pm: (pm[g], 0, 0)),
        scratch_shapes=[
            pltpu.VMEM((L, D), jnp.float32),
            pltpu.VMEM((L, D), jnp.float32),
            pltpu.SemaphoreType.DMA,
            pltpu.SemaphoreType.DMA,
        ],
    )

    return pl.pallas_call(
        _body,
        grid_spec=grid_spec,
        out_shape=jax.ShapeDtypeStruct((B, L + 1, D), vt.dtype),
    )(plen, perm, vt, emb)


# speedup vs baseline: 1.3194x; 1.0037x over previous
import jax
import jax.numpy as jnp
from jax.experimental import pallas as pl
from jax.experimental.pallas import tpu as pltpu

END_TOK = 49407


def _prefix_dma(vt_ref, buf_ref, sem, b, c8, do_start):
    for k in range(11, 2, -1):
        size = 1 << k
        off = pl.multiple_of((c8 >> (k + 1)) << (k + 1), size * 2)
        @pl.when((c8 & size) != 0)
        def _():
            dma = pltpu.make_async_copy(
                vt_ref.at[b, pl.ds(off, size)],
                buf_ref.at[pl.ds(off, size)],
                sem,
            )
            dma.start() if do_start else dma.wait()


def _body(plen_ref, vt_ref, end_ref, out_ref, buf0, buf1, sem0, sem1):
    B, L, D = vt_ref.shape
    b = pl.program_id(0)

    def c8_of(i):
        return pl.multiple_of(jnp.minimum((plen_ref[i] + 7) & ~7, L), 8)

    def stage(cur_buf, cur_sem, nxt_buf, nxt_sem):
        @pl.when(b == 0)
        def _():
            _prefix_dma(vt_ref, cur_buf, cur_sem, 0, c8_of(0), True)

        @pl.when(b + 1 < B)
        def _():
            nxt = b + 1
            _prefix_dma(vt_ref, nxt_buf, nxt_sem, nxt, c8_of(nxt), True)

        _prefix_dma(vt_ref, cur_buf, cur_sem, b, c8_of(b), False)

    plen = plen_ref[b]
    c8 = c8_of(b)
    end_row = end_ref[END_TOK % 8:END_TOK % 8 + 1, :]

    def assemble(buf):
        for k in range(11, 2, -1):
            size = 1 << k
            off = pl.multiple_of((c8 >> (k + 1)) << (k + 1), size * 2)
            @pl.when((c8 & size) != 0)
            def _():
                out_ref[0, pl.ds(off, size)] = buf[pl.ds(off, size)]

        @pl.when(c8 > plen)
        def _():
            f8 = pl.multiple_of(c8 - 8, 8)
            rows8 = jax.lax.broadcasted_iota(jnp.int32, (8, 1), 0) + f8
            tile = buf[pl.ds(f8, 8)]
            out_ref[0, pl.ds(f8, 8)] = jnp.where(rows8 < plen, tile,
                                                 end_row)

    @pl.when(b % 2 == 0)
    def _():
        stage(buf0, sem0, buf1, sem1)
        assemble(buf0)

    @pl.when(b % 2 == 1)
    def _():
        stage(buf1, sem1, buf0, sem0)
        assemble(buf1)

    q = (L - c8) >> 3
    for k in range(8, -1, -1):
        rows = 8 << k
        off = pl.multiple_of(c8 + ((q >> (k + 1)) << (k + 1)) * 8, 8)
        @pl.when((q & (1 << k)) != 0)
        def _():
            out_ref[0, pl.ds(off, rows)] = jnp.broadcast_to(
                end_row, (rows, D))
    out_ref[0, pl.ds(L, 1)] = end_row


def kernel(categories, vt, emb):
    B, L = categories.shape
    D = vt.shape[-1]
    plen = jnp.sum((categories != END_TOK).astype(jnp.int32), axis=1)

    grid_spec = pltpu.PrefetchScalarGridSpec(
        num_scalar_prefetch=1,
        grid=(B,),
        in_specs=[
            pl.BlockSpec(memory_space=pl.ANY),
            pl.BlockSpec((8, D), lambda b, p: (END_TOK // 8, 0)),
        ],
        out_specs=pl.BlockSpec((1, L + 1, D), lambda b, p: (b, 0, 0)),
        scratch_shapes=[
            pltpu.VMEM((L, D), jnp.float32),
            pltpu.VMEM((L, D), jnp.float32),
            pltpu.SemaphoreType.DMA,
            pltpu.SemaphoreType.DMA,
        ],
    )

    return pl.pallas_call(
        _body,
        grid_spec=grid_spec,
        out_shape=jax.ShapeDtypeStruct((B, L + 1, D), vt.dtype),
    )(plen, vt, emb)
